# baseline (speedup 1.0000x reference)
import functools

import jax
import jax.numpy as jnp
from jax import lax
from jax.experimental import pallas as pl
from jax.experimental.pallas import tpu as pltpu
from jax.experimental.pallas import tpu_sc as plsc

_L = 22
_N = 256
_M = 10000
_D = _L - 2
_LANES = 16
_NT = 16
_COLS = _N // _NT
_PER_TILE = _D * _COLS
_CHUNK = 80
_NCHUNK = _PER_TILE // _CHUNK
_TPC = _CHUNK // _LANES
_ROWS = (_L * _N * _M) // _LANES

_mesh = plsc.VectorSubcoreMesh(
    core_axis_name="c", subcore_axis_name="s", num_cores=1
)


@functools.partial(
    pl.kernel,
    out_type=jax.ShapeDtypeStruct((_LANES,), jnp.float32),
    mesh=_mesh,
    compiler_params=pltpu.CompilerParams(
        use_tc_tiling_on_sc=False, needs_layout_passes=False
    ),
    scratch_types=[
        pltpu.VMEM((_D, _LANES), jnp.int32),
        pltpu.VMEM((_NCHUNK, _CHUNK), jnp.int32),
        pltpu.VMEM((_D, _LANES), jnp.int32),
        pltpu.VMEM((_D, _LANES), jnp.float32),
        pltpu.VMEM((_PER_TILE, _LANES), jnp.float32),
        pltpu.VMEM((2, _LANES), jnp.float32),
        pltpu.VMEM_SHARED((_NT, 2, _LANES), jnp.float32),
        pltpu.VMEM((_NT, 2, _LANES), jnp.float32),
        pltpu.VMEM((_LANES,), jnp.float32),
        pltpu.SemaphoreType.DMA,
    ],
)
def _crit_sc(inp_hbm, seq_hbm, out_hbm, seq_v, ridx_v, lidx_v, mask_v,
             rows_v, part_v, shared, collect_v, out_v, sem):
    sid = lax.axis_index("s")
    col0 = sid * _COLS
    iota = lax.broadcasted_iota(jnp.int32, (_LANES,), 0)

    pltpu.sync_copy(seq_hbm.at[:, pl.ds(col0, _COLS)], seq_v)

    acc = jnp.zeros((_LANES,), jnp.int32)
    cnt = jnp.zeros((_LANES,), jnp.float32)
    for t in range(_D):
        row = seq_v[t, :]
        z = row == 0
        first = z & (acc == 0)
        eff = jnp.where(first, _M - 1, row)
        acc = acc + jnp.where(z, 1, 0)
        mf = jnp.where(eff != 0, 1.0, 0.0).astype(jnp.float32)
        cnt = cnt + mf
        f = (t + 1) * (_N * _M) + (col0 + iota) * _M + eff
        ridx_v[t // _TPC, pl.ds((t % _TPC) * _LANES, _LANES)] = (
            jnp.right_shift(f, 4)
        )
        lidx_v[t, :] = jnp.bitwise_and(f, _LANES - 1)
        mask_v[t, :] = mf

    copies = []
    for c in range(_NCHUNK):
        copies.append(
            pltpu.async_copy(
                inp_hbm.at[ridx_v.at[c]],
                rows_v.at[pl.ds(c * _CHUNK, _CHUNK)],
                sem,
            )
        )
    for cp in copies:
        cp.wait()

    sum_vec = jnp.zeros((_LANES,), jnp.float32)
    for t in range(_D):
        lane = lidx_v[t, :]
        vals = plsc.load_gather(rows_v, [t * _LANES + iota, lane])
        sum_vec = sum_vec + vals * mask_v[t, :]

    part_v[0, :] = sum_vec
    part_v[1, :] = cnt
    pltpu.sync_copy(part_v, shared.at[sid])
    plsc.subcore_barrier()

    @pl.when(sid == 0)
    def _():
        pltpu.sync_copy(shared, collect_v)
        ts = jnp.zeros((_LANES,), jnp.float32)
        tc = jnp.zeros((_LANES,), jnp.float32)
        for i in range(_NT):
            ts = ts + collect_v[i, 0, :]
            tc = tc + collect_v[i, 1, :]
        out_v[:] = -(plsc.cumsum(ts) / plsc.cumsum(tc))
        pltpu.sync_copy(out_v, out_hbm)


def kernel(input, seq):
    table = input.reshape(_ROWS, _LANES)
    out = _crit_sc(table, seq)
    return out[_LANES - 1]


# speedup vs baseline: 1.5467x; 1.5467x over previous
import functools

import jax
import jax.numpy as jnp
from jax import lax
from jax.experimental import pallas as pl
from jax.experimental.pallas import tpu as pltpu
from jax.experimental.pallas import tpu_sc as plsc

_L = 22
_N = 256
_M = 10000
_D = _L - 2
_LANES = 16
_NT = 16
_COLS = _N // _NT

_mesh = plsc.VectorSubcoreMesh(
    core_axis_name="c", subcore_axis_name="s", num_cores=1
)


@functools.partial(
    pl.kernel,
    out_type=(
        jax.ShapeDtypeStruct((_NT, 2, _LANES), jnp.float32),
        jax.ShapeDtypeStruct((_LANES,), jnp.float32),
    ),
    mesh=_mesh,
    compiler_params=pltpu.CompilerParams(needs_layout_passes=False),
    scratch_types=[
        pltpu.VMEM((_D * _N,), jnp.int32),
        pltpu.VMEM((_D, _LANES), jnp.int32),
        pltpu.VMEM((_D, _LANES), jnp.int32),
        pltpu.VMEM((_D, _LANES), jnp.float32),
        pltpu.VMEM((_LANES, 8, 128), jnp.float32),
        pltpu.VMEM((2, _LANES), jnp.float32),
        pltpu.VMEM((_NT, 2, _LANES), jnp.float32),
        pltpu.VMEM((_LANES,), jnp.float32),
        pltpu.VMEM((_LANES,), jnp.float32),
        pltpu.SemaphoreType.DMA,
    ],
)
def _crit_sc(inp_hbm, seq_hbm, part_hbm, loss_hbm, seq_v, eff_v, lidx_v,
             mask_v, buf, part_v, collect_v, out_v, sums_v, sem):
    sid = lax.axis_index("s")
    col0 = pl.multiple_of(sid * _COLS, _COLS)
    iota = lax.broadcasted_iota(jnp.int32, (_LANES,), 0)

    pltpu.sync_copy(seq_hbm, seq_v)

    acc = jnp.zeros((_LANES,), jnp.int32)
    cnt = jnp.zeros((_LANES,), jnp.float32)
    for t in range(_D):
        row = seq_v[pl.ds(t * _N + col0, _COLS)]
        z = row == 0
        first = z & (acc == 0)
        eff = jnp.where(first, _M - 1, row)
        acc = acc + jnp.where(z, 1, 0)
        mf = jnp.where(eff != 0, 1.0, 0.0).astype(jnp.float32)
        cnt = cnt + mf
        eff_v[t, :] = eff
        lidx_v[t, :] = jnp.bitwise_and(eff, 127)
        mask_v[t, :] = mf
    part_v[1, :] = cnt

    sums_v[:] = jnp.zeros((_LANES,), jnp.float32)

    def body(t, s):
        eff_row = eff_v[t, :]
        copies = []
        for j in range(_COLS):
            e = eff_row[j]
            v0 = pl.multiple_of(jnp.right_shift(e, 7) * 128, 128)
            b0 = pl.multiple_of(col0 + (j & ~7), 8)
            src = inp_hbm.at[t + 1, pl.ds(b0, 8), pl.ds(v0, 128)]
            copies.append(pltpu.async_copy(src, buf.at[j], sem))
        for cp in copies:
            cp.wait()
        subrow = jnp.bitwise_and(iota, 7)
        vals = plsc.load_gather(buf, [iota, subrow, lidx_v[t, :]])
        sums_v[:] = sums_v[:] + vals * mask_v[t, :]
        return s + 1

    lax.fori_loop(0, _D, body, jnp.int32(0))

    part_v[0, :] = sums_v[:]
    pltpu.sync_copy(part_v, part_hbm.at[sid])
    plsc.subcore_barrier()

    @pl.when(sid == 0)
    def _():
        pltpu.sync_copy(part_hbm, collect_v)
        ts = jnp.zeros((_LANES,), jnp.float32)
        tc = jnp.zeros((_LANES,), jnp.float32)
        for i in range(_NT):
            ts = ts + collect_v[i, 0, :]
            tc = tc + collect_v[i, 1, :]
        out_v[:] = -(plsc.cumsum(ts) / plsc.cumsum(tc))
        pltpu.sync_copy(out_v, loss_hbm)


def kernel(input, seq):
    _, loss = _crit_sc(input, seq.reshape(-1))
    return loss[_LANES - 1]
